# Initial kernel scaffold; baseline (speedup 1.0000x reference)
import jax
import jax.numpy as jnp
from jax.experimental import pallas as pl


def kernel(z_pred, expert_eligibility, W1, b1, W2, b2, W3, b3):
    raise NotImplementedError("write your pallas kernel here")


# speedup vs baseline: 1.5269x; 1.5269x over previous
import functools

import jax
import jax.numpy as jnp
from jax import lax
from jax.experimental import pallas as pl
from jax.experimental.pallas import tpu as pltpu

TOPK = 2
BLK = 256
HT = 2


def _moe_body(m_ref, x_ref, w1_ref, b1_ref, w2_ref, b2_ref, w3_ref, b3_ref,
              gt_ref, out_ref, h_scr, *, nb):
    g = pl.program_id(0)
    t = pl.program_id(1)

    @pl.when(g < m_ref[nb])
    def _():
        @pl.when(t == 0)
        def _():
            h = jnp.dot(x_ref[...], w1_ref[0],
                        preferred_element_type=jnp.float32) + b1_ref[0]
            h_scr[...] = h * 0.5 * (1.0 + lax.erf(h * 0.7071067811865476))

        h2 = jnp.dot(h_scr[...], w2_ref[0],
                     preferred_element_type=jnp.float32) + b2_ref[0]
        part = jnp.dot(h2, w3_ref[0], preferred_element_type=jnp.float32)

        @pl.when(t == 0)
        def _():
            out_ref[...] = (part + b3_ref[0]) * gt_ref[...]

        @pl.when(t != 0)
        def _():
            out_ref[...] += part * gt_ref[...]


def kernel(z_pred, expert_eligibility, W1, b1, W2, b2, W3, b3):
    n, d = z_pred.shape
    e = W1.shape[0]
    h_dim = W1.shape[2]
    ht = h_dim // HT
    np_ = n * TOPK
    nb = np_ // BLK + e - 1
    npad = nb * BLK

    vals, idx = lax.top_k(expert_eligibility, TOPK)
    gates = jax.nn.softmax(vals, axis=-1)
    e_flat = idx.reshape(-1).astype(jnp.int32)
    g_flat = gates.reshape(-1)

    order = jnp.argsort(e_flat, stable=True)
    counts = jnp.bincount(e_flat, length=e).astype(jnp.int32)
    offs = jnp.concatenate([jnp.zeros(1, jnp.int32),
                            jnp.cumsum(counts)])[:e]
    nblk_e = (counts + BLK - 1) // BLK
    cum_blocks = jnp.cumsum(nblk_e)
    total_blocks = cum_blocks[-1]
    poffs = jnp.concatenate([jnp.zeros(1, jnp.int32),
                             jnp.cumsum(nblk_e * BLK)])[:e]

    bids = jnp.arange(nb, dtype=jnp.int32)
    block_expert = jnp.searchsorted(
        cum_blocks, jnp.minimum(bids, total_blocks - 1), side="right"
    ).astype(jnp.int32)
    meta = jnp.concatenate([block_expert, total_blocks[None]])

    q = jnp.arange(npad, dtype=jnp.int32)
    eq = block_expert[q // BLK]
    r = q - poffs[eq]
    valid = r < counts[eq]
    src_pair = order[offs[eq] + jnp.minimum(r, counts[eq] - 1)]
    tok_src = (src_pair // TOPK).astype(jnp.int32)
    gate_col = jnp.where(valid, g_flat[src_pair], 0.0).reshape(npad, 1)

    x_sorted = z_pred[tok_src]

    grid_spec = pltpu.PrefetchScalarGridSpec(
        num_scalar_prefetch=1,
        grid=(nb, HT),
        in_specs=[
            pl.BlockSpec((BLK, d), lambda g, t, m: (g, 0)),
            pl.BlockSpec((1, d, h_dim), lambda g, t, m: (m[g], 0, 0)),
            pl.BlockSpec((1, 1, h_dim), lambda g, t, m: (m[g], 0, 0)),
            pl.BlockSpec((1, h_dim, ht), lambda g, t, m: (m[g], 0, t)),
            pl.BlockSpec((1, 1, ht), lambda g, t, m: (m[g], 0, t)),
            pl.BlockSpec((1, ht, d), lambda g, t, m: (m[g], t, 0)),
            pl.BlockSpec((1, 1, d), lambda g, t, m: (m[g], 0, 0)),
            pl.BlockSpec((BLK, 1), lambda g, t, m: (g, 0)),
        ],
        out_specs=pl.BlockSpec((BLK, d), lambda g, t, m: (g, 0)),
        scratch_shapes=[pltpu.VMEM((BLK, h_dim), jnp.float32)],
    )
    out_rows = pl.pallas_call(
        functools.partial(_moe_body, nb=nb),
        grid_spec=grid_spec,
        out_shape=jax.ShapeDtypeStruct((npad, d), jnp.float32),
        compiler_params=pltpu.CompilerParams(
            dimension_semantics=("arbitrary", "arbitrary")),
    )(meta, x_sorted, W1, b1.reshape(e, 1, h_dim), W2,
      b2.reshape(e, 1, h_dim), W3, b3.reshape(e, 1, d), gate_col)

    pos = jnp.zeros(np_, jnp.int32).at[order].set(
        jnp.arange(np_, dtype=jnp.int32))
    padded_pos = pos - offs[e_flat] + poffs[e_flat]
    y = out_rows[padded_pos.reshape(n, TOPK)].sum(axis=1)
    return y
